# baseline (speedup 1.0000x reference)
import functools

import numpy as np
import jax
import jax.numpy as jnp
from jax import lax
from jax.experimental import pallas as pl
from jax.experimental.pallas import tpu as pltpu
from jax.experimental.pallas import tpu_sc as plsc

NC = 2
NS = 16
NW = NC * NS
LANES = 16
UNROLL = 8
CS = 16
NRB = 4


def _pos_encoding(seq_len, d_model):
    pos = np.arange(seq_len)[:, None].astype(np.float32)
    i = np.arange(0, d_model, 2).astype(np.float32)
    angle = pos / np.power(10000.0, i / d_model)
    pe = np.zeros((seq_len, d_model), dtype=np.float32)
    pe[:, 0::2] = np.sin(angle)
    pe[:, 1::2] = np.cos(angle)
    return pe


@functools.cache
def _build(batch, seq, vocab, d):
    spw = seq // NW
    nchunk = spw // CS
    nvec = d // LANES
    assert spw * NW == seq and nchunk * CS == spw
    assert nvec % UNROLL == 0 and batch == NRB and nchunk % 2 == 0
    ngp = nchunk // 2
    ntask = nchunk * batch

    mesh = plsc.VectorSubcoreMesh(
        core_axis_name="c", subcore_axis_name="s",
        num_cores=NC, num_subcores=NS)

    @functools.partial(
        pl.kernel,
        mesh=mesh,
        out_type=jax.ShapeDtypeStruct((batch * seq, d), jnp.float32),
        scratch_types=[
            pltpu.VMEM((batch * spw,), jnp.int32),
            [pltpu.VMEM((CS, d), jnp.float32) for _ in range(2)],
            [pltpu.VMEM((CS, d), jnp.float32) for _ in range(NRB)],
            pltpu.SemaphoreType.DMA,
            pltpu.SemaphoreType.DMA,
            pltpu.SemaphoreType.DMA,
        ],
    )
    def emb(x_hbm, tab_hbm, pe_hbm, out_hbm, idx_v, pes, rows,
            sem_pe, sem_g, sem_out):
        wid = lax.axis_index("s") * NC + lax.axis_index("c")
        sw = wid * spw

        for b in range(batch):
            pltpu.sync_copy(x_hbm.at[b, pl.ds(sw, spw)],
                            idx_v.at[pl.ds(b * spw, spw)])

        def gather(g, b, rb):
            pltpu.async_copy(
                tab_hbm.at[idx_v.at[pl.ds(b * spw + g * CS, CS)]],
                rb, sem_g)

        def wait_gather(rb):
            pltpu.make_async_copy(
                tab_hbm.at[idx_v.at[pl.ds(0, CS)]], rb, sem_g).wait()

        def wait_out(rb):
            pltpu.make_async_copy(rb, out_hbm.at[pl.ds(0, CS)],
                                  sem_out).wait()

        def add_chunk(rb, pb):
            @plsc.parallel_loop(0, CS * nvec, step=1, unroll=UNROLL)
            def _(i):
                r = i // nvec
                o = (i % nvec) * LANES
                plsc.addupdate(rb.at[r, pl.ds(o, LANES)],
                               pb[r, pl.ds(o, LANES)])

        pltpu.async_copy(pe_hbm.at[pl.ds(sw, CS)], pes[0], sem_pe)
        gather(0, 0, rows[0])

        def body(gp, carry):
            for gg in range(2):
                g = gp * 2 + gg
                pb = pes[gg]
                for b in range(batch):
                    rb = rows[b]
                    nrb = rows[(b + 1) % NRB]
                    if gg == 0 and b < 3:
                        @pl.when(gp > 0)
                        def _():
                            wait_out(nrb)
                        if b < batch - 1:
                            gather(g, b + 1, nrb)
                        else:
                            gather(g + 1, 0, nrb)
                    elif gg == 1 and b == 3:
                        wait_out(nrb)
                        @pl.when(gp < ngp - 1)
                        def _():
                            gather(g + 1, 0, nrb)
                    else:
                        wait_out(nrb)
                        if b < batch - 1:
                            gather(g, b + 1, nrb)
                        else:
                            gather(g + 1, 0, nrb)
                    if b == 0:
                        pltpu.make_async_copy(
                            pe_hbm.at[pl.ds(0, CS)], pb, sem_pe).wait()
                        if gg == 0:
                            pltpu.async_copy(
                                pe_hbm.at[pl.ds(sw + (g + 1) * CS, CS)],
                                pes[1 - gg], sem_pe)
                        else:
                            @pl.when(gp < ngp - 1)
                            def _():
                                pltpu.async_copy(
                                    pe_hbm.at[pl.ds(sw + (g + 1) * CS, CS)],
                                    pes[1 - gg], sem_pe)
                    wait_gather(rb)
                    add_chunk(rb, pb)
                    pltpu.async_copy(
                        rb, out_hbm.at[pl.ds(b * seq + sw + g * CS, CS)],
                        sem_out)
            return carry

        lax.fori_loop(0, ngp, body, 0)

        for k in range(NRB - 1):
            wait_out(rows[(ntask - (NRB - 1) + k) % NRB])

    return emb


def kernel(x, token_table):
    batch, seq = x.shape
    vocab, d = token_table.shape
    pe = jnp.asarray(_pos_encoding(seq, d))
    xi = x.astype(jnp.int32)
    out = _build(batch, seq, vocab, d)(xi, token_table, pe)
    return out.reshape(batch, seq, d)


# speedup vs baseline: 1.0454x; 1.0454x over previous
import functools

import numpy as np
import jax
import jax.numpy as jnp
from jax import lax
from jax.experimental import pallas as pl
from jax.experimental.pallas import tpu as pltpu
from jax.experimental.pallas import tpu_sc as plsc

NC = 2
NS = 16
NW = NC * NS
LANES = 16
UNROLL = 8
CS = 16
NRB = 4


def _pos_encoding(seq_len, d_model):
    pos = np.arange(seq_len)[:, None].astype(np.float32)
    i = np.arange(0, d_model, 2).astype(np.float32)
    angle = pos / np.power(10000.0, i / d_model)
    pe = np.zeros((seq_len, d_model), dtype=np.float32)
    pe[:, 0::2] = np.sin(angle)
    pe[:, 1::2] = np.cos(angle)
    return pe


@functools.cache
def _build(batch, seq, vocab, d):
    spw = seq // NW
    nchunk = spw // CS
    nvec = d // LANES
    assert spw * NW == seq and nchunk * CS == spw
    assert nvec % UNROLL == 0 and batch == NRB and nchunk % 2 == 0
    ngp = nchunk // 2
    ntask = nchunk * batch

    mesh = plsc.VectorSubcoreMesh(
        core_axis_name="c", subcore_axis_name="s",
        num_cores=NC, num_subcores=NS)

    @functools.partial(
        pl.kernel,
        mesh=mesh,
        out_type=jax.ShapeDtypeStruct((batch * seq, d), jnp.float32),
        scratch_types=[
            pltpu.VMEM((batch * spw,), jnp.int32),
            [pltpu.VMEM((CS, d), jnp.float32) for _ in range(2)],
            [pltpu.VMEM((CS, d), jnp.float32) for _ in range(NRB)],
            pltpu.SemaphoreType.DMA,
            pltpu.SemaphoreType.DMA,
            pltpu.SemaphoreType.DMA,
        ],
    )
    def emb(x_hbm, tab_hbm, pe_hbm, out_hbm, idx_v, pes, rows,
            sem_pe, sem_g, sem_out):
        wid = lax.axis_index("s") * NC + lax.axis_index("c")
        sw = wid * spw

        for b in range(batch):
            pltpu.sync_copy(x_hbm.at[b, pl.ds(sw, spw)],
                            idx_v.at[pl.ds(b * spw, spw)])

        def gather(g, b, rb):
            pltpu.async_copy(
                tab_hbm.at[idx_v.at[pl.ds(b * spw + g * CS, CS)]],
                rb, sem_g)

        def wait_gather(rb):
            pltpu.make_async_copy(
                tab_hbm.at[idx_v.at[pl.ds(0, CS)]], rb, sem_g).wait()

        def wait_out(rb):
            pltpu.make_async_copy(rb, out_hbm.at[pl.ds(0, CS)],
                                  sem_out).wait()

        def add_chunk(rb, pb):
            @plsc.parallel_loop(0, CS * nvec, step=1, unroll=UNROLL)
            def _(i):
                r = i // nvec
                o = (i % nvec) * LANES
                plsc.addupdate(rb.at[r, pl.ds(o, LANES)],
                               pb[r, pl.ds(o, LANES)])

        pltpu.async_copy(pe_hbm.at[pl.ds(sw, CS)], pes[0], sem_pe)
        gather(0, 0, rows[0])
        gather(0, 1, rows[1])

        def body(gp, carry):
            for gg in range(2):
                g = gp * 2 + gg
                pb = pes[gg]
                for b in range(batch):
                    rb = rows[b]
                    nrb = rows[(b + 2) % NRB]
                    if b < 2:
                        ng, nb = g, b + 2
                    else:
                        ng, nb = g + 1, b - 2
                    if gg == 0 and b < 2:
                        @pl.when(gp > 0)
                        def _():
                            wait_out(nrb)
                        gather(ng, nb, nrb)
                    elif gg == 1 and b >= 2:
                        wait_out(nrb)
                        @pl.when(gp < ngp - 1)
                        def _():
                            gather(ng, nb, nrb)
                    else:
                        wait_out(nrb)
                        gather(ng, nb, nrb)
                    if b == 0:
                        pltpu.make_async_copy(
                            pe_hbm.at[pl.ds(0, CS)], pb, sem_pe).wait()
                        if gg == 0:
                            pltpu.async_copy(
                                pe_hbm.at[pl.ds(sw + (g + 1) * CS, CS)],
                                pes[1 - gg], sem_pe)
                        else:
                            @pl.when(gp < ngp - 1)
                            def _():
                                pltpu.async_copy(
                                    pe_hbm.at[pl.ds(sw + (g + 1) * CS, CS)],
                                    pes[1 - gg], sem_pe)
                    wait_gather(rb)
                    add_chunk(rb, pb)
                    pltpu.async_copy(
                        rb, out_hbm.at[pl.ds(b * seq + sw + g * CS, CS)],
                        sem_out)
            return carry

        lax.fori_loop(0, ngp, body, 0)

        for k in range(2):
            wait_out(rows[(ntask - 2 + k) % NRB])

    return emb


def kernel(x, token_table):
    batch, seq = x.shape
    vocab, d = token_table.shape
    pe = jnp.asarray(_pos_encoding(seq, d))
    xi = x.astype(jnp.int32)
    out = _build(batch, seq, vocab, d)(xi, token_table, pe)
    return out.reshape(batch, seq, d)


# speedup vs baseline: 1.0654x; 1.0191x over previous
import functools

import numpy as np
import jax
import jax.numpy as jnp
from jax import lax
from jax.experimental import pallas as pl
from jax.experimental.pallas import tpu as pltpu
from jax.experimental.pallas import tpu_sc as plsc

NC = 2
NS = 16
NW = NC * NS
LANES = 16
UNROLL = 8
CS = 16
NRB = 4


def _pos_encoding(seq_len, d_model):
    pos = np.arange(seq_len)[:, None].astype(np.float32)
    i = np.arange(0, d_model, 2).astype(np.float32)
    angle = pos / np.power(10000.0, i / d_model)
    pe = np.zeros((seq_len, d_model), dtype=np.float32)
    pe[:, 0::2] = np.sin(angle)
    pe[:, 1::2] = np.cos(angle)
    return pe


@functools.cache
def _build(batch, seq, vocab, d):
    spw = seq // NW
    nchunk = spw // CS
    nvec = d // LANES
    assert spw * NW == seq and nchunk * CS == spw
    assert nvec % UNROLL == 0 and batch == NRB and nchunk % 2 == 0
    ngp = nchunk // 2
    ntask = nchunk * batch

    mesh = plsc.VectorSubcoreMesh(
        core_axis_name="c", subcore_axis_name="s",
        num_cores=NC, num_subcores=NS)

    @functools.partial(
        pl.kernel,
        mesh=mesh,
        out_type=jax.ShapeDtypeStruct((batch * seq, d), jnp.float32),
        scratch_types=[
            pltpu.VMEM((batch * spw,), jnp.int32),
            [pltpu.VMEM((CS, d), jnp.float32) for _ in range(2)],
            [pltpu.VMEM((CS, d), jnp.float32) for _ in range(NRB)],
            pltpu.SemaphoreType.DMA,
            pltpu.SemaphoreType.DMA,
            pltpu.SemaphoreType.DMA,
        ],
    )
    def emb(x_hbm, tab_hbm, pe_hbm, out_hbm, idx_v, pes, rows,
            sem_pe, sem_g, sem_out):
        wid = lax.axis_index("s") * NC + lax.axis_index("c")
        sw = wid * spw

        idx_cps = [
            pltpu.async_copy(x_hbm.at[b, pl.ds(sw, spw)],
                             idx_v.at[pl.ds(b * spw, spw)], sem_out)
            for b in range(batch)
        ]
        for cp in idx_cps:
            cp.wait()

        def gather(g, b, rb):
            pltpu.async_copy(
                tab_hbm.at[idx_v.at[pl.ds(b * spw + g * CS, CS)]],
                rb, sem_g)

        def wait_gather(rb):
            pltpu.make_async_copy(
                tab_hbm.at[idx_v.at[pl.ds(0, CS)]], rb, sem_g).wait()

        def wait_out(rb):
            pltpu.make_async_copy(rb, out_hbm.at[pl.ds(0, CS)],
                                  sem_out).wait()

        def add_chunk(rb, pb):
            @plsc.parallel_loop(0, CS * nvec, step=1, unroll=UNROLL)
            def _(i):
                r = i // nvec
                o = (i % nvec) * LANES
                plsc.addupdate(rb.at[r, pl.ds(o, LANES)],
                               pb[r, pl.ds(o, LANES)])

        pltpu.async_copy(pe_hbm.at[pl.ds(sw, CS)], pes[0], sem_pe)
        gather(0, 0, rows[0])
        gather(0, 1, rows[1])

        def body(gp, carry):
            for gg in range(2):
                g = gp * 2 + gg
                pb = pes[gg]
                for b in range(batch):
                    rb = rows[b]
                    nrb = rows[(b + 2) % NRB]
                    if b < 2:
                        ng, nb = g, b + 2
                    else:
                        ng, nb = g + 1, b - 2
                    if gg == 0 and b < 2:
                        @pl.when(gp > 0)
                        def _():
                            wait_out(nrb)
                        gather(ng, nb, nrb)
                    elif gg == 1 and b >= 2:
                        wait_out(nrb)
                        @pl.when(gp < ngp - 1)
                        def _():
                            gather(ng, nb, nrb)
                    else:
                        wait_out(nrb)
                        gather(ng, nb, nrb)
                    if b == 0:
                        pltpu.make_async_copy(
                            pe_hbm.at[pl.ds(0, CS)], pb, sem_pe).wait()
                        if gg == 0:
                            pltpu.async_copy(
                                pe_hbm.at[pl.ds(sw + (g + 1) * CS, CS)],
                                pes[1 - gg], sem_pe)
                        else:
                            @pl.when(gp < ngp - 1)
                            def _():
                                pltpu.async_copy(
                                    pe_hbm.at[pl.ds(sw + (g + 1) * CS, CS)],
                                    pes[1 - gg], sem_pe)
                    wait_gather(rb)
                    add_chunk(rb, pb)
                    pltpu.async_copy(
                        rb, out_hbm.at[pl.ds(b * seq + sw + g * CS, CS)],
                        sem_out)
            return carry

        lax.fori_loop(0, ngp, body, 0)

        for k in range(2):
            wait_out(rows[(ntask - 2 + k) % NRB])

    return emb


def kernel(x, token_table):
    batch, seq = x.shape
    vocab, d = token_table.shape
    pe = jnp.asarray(_pos_encoding(seq, d))
    xi = x.astype(jnp.int32)
    out = _build(batch, seq, vocab, d)(xi, token_table, pe)
    return out.reshape(batch, seq, d)


# speedup vs baseline: 1.0708x; 1.0051x over previous
import functools

import numpy as np
import jax
import jax.numpy as jnp
from jax import lax
from jax.experimental import pallas as pl
from jax.experimental.pallas import tpu as pltpu
from jax.experimental.pallas import tpu_sc as plsc

NC = 2
NS = 16
NW = NC * NS
LANES = 16
UNROLL = 8
CS = 8
NRB = 8


def _pos_encoding(seq_len, d_model):
    pos = np.arange(seq_len)[:, None].astype(np.float32)
    i = np.arange(0, d_model, 2).astype(np.float32)
    angle = pos / np.power(10000.0, i / d_model)
    pe = np.zeros((seq_len, d_model), dtype=np.float32)
    pe[:, 0::2] = np.sin(angle)
    pe[:, 1::2] = np.cos(angle)
    return pe


@functools.cache
def _build(batch, seq, vocab, d):
    spw = seq // NW
    nchunk = spw // CS
    nvec = d // LANES
    assert spw * NW == seq and nchunk * CS == spw
    assert nvec % UNROLL == 0 and 2 * batch == NRB and nchunk % 2 == 0
    ngp = nchunk // 2
    ntask = nchunk * batch

    mesh = plsc.VectorSubcoreMesh(
        core_axis_name="c", subcore_axis_name="s",
        num_cores=NC, num_subcores=NS)

    @functools.partial(
        pl.kernel,
        mesh=mesh,
        out_type=jax.ShapeDtypeStruct((batch * seq, d), jnp.float32),
        scratch_types=[
            pltpu.VMEM((batch * spw,), jnp.int32),
            [pltpu.VMEM((CS, d), jnp.float32) for _ in range(2)],
            [pltpu.VMEM((CS, d), jnp.float32) for _ in range(NRB)],
            pltpu.SemaphoreType.DMA,
            pltpu.SemaphoreType.DMA,
            pltpu.SemaphoreType.DMA,
        ],
    )
    def emb(x_hbm, tab_hbm, pe_hbm, out_hbm, idx_v, pes, rows,
            sem_pe, sem_g, sem_out):
        wid = lax.axis_index("s") * NC + lax.axis_index("c")
        sw = wid * spw

        idx_cps = [
            pltpu.async_copy(x_hbm.at[b, pl.ds(sw, spw)],
                             idx_v.at[pl.ds(b * spw, spw)], sem_out)
            for b in range(batch)
        ]
        for cp in idx_cps:
            cp.wait()

        def gather(g, b, rb):
            pltpu.async_copy(
                tab_hbm.at[idx_v.at[pl.ds(b * spw + g * CS, CS)]],
                rb, sem_g)

        def wait_gather(rb):
            pltpu.make_async_copy(
                tab_hbm.at[idx_v.at[pl.ds(0, CS)]], rb, sem_g).wait()

        def wait_out(rb):
            pltpu.make_async_copy(rb, out_hbm.at[pl.ds(0, CS)],
                                  sem_out).wait()

        def add_chunk(rb, pb):
            @plsc.parallel_loop(0, CS * nvec, step=1, unroll=UNROLL)
            def _(i):
                r = i // nvec
                o = (i % nvec) * LANES
                plsc.addupdate(rb.at[r, pl.ds(o, LANES)],
                               pb[r, pl.ds(o, LANES)])

        pltpu.async_copy(pe_hbm.at[pl.ds(sw, CS)], pes[0], sem_pe)
        for b in range(batch):
            gather(0, b, rows[b])

        def body(gp, carry):
            for gg in range(2):
                g = gp * 2 + gg
                pb = pes[gg]
                for b in range(batch):
                    rb = rows[batch * gg + b]
                    nrb = rows[batch * (1 - gg) + b]
                    if gg == 0:
                        @pl.when(gp > 0)
                        def _():
                            wait_out(nrb)
                        gather(g + 1, b, nrb)
                    else:
                        wait_out(nrb)
                        @pl.when(gp < ngp - 1)
                        def _():
                            gather(g + 1, b, nrb)
                    if b == 0:
                        pltpu.make_async_copy(
                            pe_hbm.at[pl.ds(0, CS)], pb, sem_pe).wait()
                        if gg == 0:
                            pltpu.async_copy(
                                pe_hbm.at[pl.ds(sw + (g + 1) * CS, CS)],
                                pes[1 - gg], sem_pe)
                        else:
                            @pl.when(gp < ngp - 1)
                            def _():
                                pltpu.async_copy(
                                    pe_hbm.at[pl.ds(sw + (g + 1) * CS, CS)],
                                    pes[1 - gg], sem_pe)
                    wait_gather(rb)
                    add_chunk(rb, pb)
                    pltpu.async_copy(
                        rb, out_hbm.at[pl.ds(b * seq + sw + g * CS, CS)],
                        sem_out)
            return carry

        lax.fori_loop(0, ngp, body, 0)

        for k in range(batch):
            wait_out(rows[(ntask - batch + k) % NRB])

    return emb


def kernel(x, token_table):
    batch, seq = x.shape
    vocab, d = token_table.shape
    pe = jnp.asarray(_pos_encoding(seq, d))
    xi = x.astype(jnp.int32)
    out = _build(batch, seq, vocab, d)(xi, token_table, pe)
    return out.reshape(batch, seq, d)
